# Initial kernel scaffold; baseline (speedup 1.0000x reference)
import jax
import jax.numpy as jnp
from jax.experimental import pallas as pl


def kernel(coords, feats, mc_points, gib_params_0, cvx_0, gib_params_1, cvx_1, W, b):
    raise NotImplementedError("write your pallas kernel here")


# speedup vs baseline: 7.1909x; 7.1909x over previous
import functools

import jax
import jax.numpy as jnp
from jax.experimental import pallas as pl

_N = 4096
_Q = 512
_K = 16
_M = 1000
_MC_PAD = 1024
_G = 8
_INV_DENOM = 0.08


def _gib_knn_body(cs_ref, cq_ref, mc_ref, gp0_ref, cvx0_ref, gp1_ref,
                  cvx1_ref, wt_ref, b_ref, out_ref):
    cs = cs_ref[0]
    xs = cs[:, 0:1]
    ys = cs[:, 1:2]
    zs = cs[:, 2:3]
    cq = cq_ref[0]
    xq = cq[0:1, :]
    yq = cq[1:2, :]
    zq = cq[2:3, :]

    def _b(v):
        return v.astype(jnp.bfloat16).astype(jnp.float32)

    sq_s = xs * xs + ys * ys + zs * zs
    sq_q = xq * xq + yq * yq + zq * zq
    dot = (_b(xs) * _b(xq) + _b(ys) * _b(yq) + _b(zs) * _b(zq))
    d2 = sq_s + sq_q - 2.0 * dot

    iota_s = jax.lax.broadcasted_iota(jnp.int32, (_N, _Q), 0)

    gp0 = gp0_ref[...]
    g0x, g0y, g0z = gp0[:, 0:1], gp0[:, 1:2], gp0[:, 2:3]
    gp1 = gp1_ref[...]
    g1x, g1y, g1z = gp1[:, 0:1], gp1[:, 1:2], gp1[:, 2:3]

    resp0 = jnp.zeros((_G, _Q), jnp.float32)
    resp1 = jnp.zeros((_G, _Q), jnp.float32)

    for t in range(_K):
        m = jnp.min(d2, axis=0, keepdims=True)
        am = jnp.min(jnp.where(d2 == m, iota_s, _N),
                     axis=0, keepdims=True)
        onehot = iota_s == am
        nbx = jnp.sum(jnp.where(onehot, xs, 0.0), axis=0, keepdims=True)
        nby = jnp.sum(jnp.where(onehot, ys, 0.0), axis=0, keepdims=True)
        nbz = jnp.sum(jnp.where(onehot, zs, 0.0), axis=0, keepdims=True)
        d2 = jnp.where(onehot, jnp.inf, d2)
        relx = nbx - xq
        rely = nby - yq
        relz = nbz - zq
        if t < 8:
            s0x = g0x * relx
            s0y = g0y * rely
            s0z = g0z * relz
            qf0 = s0x * s0x + s0y * s0y + s0z * s0z
            resp0 = resp0 + jnp.exp(-qf0 / _INV_DENOM)
        s1x = g1x * relx
        s1y = g1y * rely
        s1z = g1z * relz
        qf1 = s1x * s1x + s1y * s1y + s1z * s1z
        resp1 = resp1 + jnp.exp(-qf1 / _INV_DENOM)

    mcx = mc_ref[0:1, :]
    mcy = mc_ref[1:2, :]
    mcz = mc_ref[2:3, :]
    lane = jax.lax.broadcasted_iota(jnp.int32, (_G, _MC_PAD), 1)
    valid = lane < _M

    def integ(gx, gy, gz):
        ax = gx * mcx
        ay = gy * mcy
        az = gz * mcz
        acc = ax * ax + ay * ay + az * az
        e = jnp.where(valid, jnp.exp(-acc / _INV_DENOM), 0.0)
        return jnp.sum(e, axis=1, keepdims=True) / float(_M)

    integ0 = integ(g0x, g0y, g0z)
    integ1 = integ(g1x, g1y, g1z)
    resp0n = resp0 / (integ0 + 1e-8)
    resp1n = resp1 / (integ1 + 1e-8)

    w0 = jax.nn.softmax(cvx0_ref[...], axis=-1)
    w1 = jax.nn.softmax(cvx1_ref[...], axis=-1)
    out0 = jnp.dot(w0, resp0n, preferred_element_type=jnp.float32)
    out1 = 0.5 * jnp.dot(w1, resp1n, preferred_element_type=jnp.float32)
    out_cat = jnp.concatenate([out0, out1], axis=0)
    x = jnp.dot(wt_ref[...], out_cat, preferred_element_type=jnp.float32)
    x = jnp.maximum(x + b_ref[...], 0.0)
    out_ref[0] = jnp.concatenate([x, out_cat], axis=0)


@functools.partial(jax.jit, static_argnames=())
def kernel(coords, feats, mc_points, gib_params_0, cvx_0, gib_params_1,
           cvx_1, W, b):
    del feats
    B, N, _ = coords.shape
    coords_t = jnp.transpose(coords, (0, 2, 1))
    mc_t = jnp.pad(mc_points, ((0, _MC_PAD - _M), (0, 0))).T
    w_t = W.T
    b_col = b[:, None]

    grid = (B, N // _Q)
    out_t = pl.pallas_call(
        _gib_knn_body,
        grid=grid,
        in_specs=[
            pl.BlockSpec((1, N, 3), lambda bi, qi: (bi, 0, 0)),
            pl.BlockSpec((1, 3, _Q), lambda bi, qi: (bi, 0, qi)),
            pl.BlockSpec((3, _MC_PAD), lambda bi, qi: (0, 0)),
            pl.BlockSpec((_G, 3), lambda bi, qi: (0, 0)),
            pl.BlockSpec(cvx_0.shape, lambda bi, qi: (0, 0)),
            pl.BlockSpec((_G, 3), lambda bi, qi: (0, 0)),
            pl.BlockSpec(cvx_1.shape, lambda bi, qi: (0, 0)),
            pl.BlockSpec(w_t.shape, lambda bi, qi: (0, 0)),
            pl.BlockSpec(b_col.shape, lambda bi, qi: (0, 0)),
        ],
        out_specs=pl.BlockSpec((1, 48, _Q), lambda bi, qi: (bi, 0, qi)),
        out_shape=jax.ShapeDtypeStruct((B, 48, N), jnp.float32),
    )(coords, coords_t, mc_t, gib_params_0, cvx_0, gib_params_1, cvx_1,
      w_t, b_col)
    return jnp.transpose(out_t, (0, 2, 1))
